# Initial kernel scaffold; baseline (speedup 1.0000x reference)
import jax
import jax.numpy as jnp
from jax.experimental import pallas as pl


def kernel(x, edge_index_train, edge_pairs, W1, b1, W2, b2, lpW1, lpb1, lpW2, lpb2):
    raise NotImplementedError("write your pallas kernel here")


# speedup vs baseline: 6.7743x; 6.7743x over previous
import functools

import jax
import jax.numpy as jnp
from jax import lax
from jax.experimental import pallas as pl
from jax.experimental.pallas import tpu as pltpu
from jax.experimental.pallas import tpu_sc as plsc

N = 10000
D = 128
H = 128
E = 320000
P = 65536

NC = 2
NS = 16
NT = NC * NS
NPAD = 10240
ROWS_T = NPAD // NS
EB = 128
EPT = 10240
NB_E = EPT // EB
EPAD = NT * EPT
PB = (2 * P) // NT
NB_P = PB // EB

_mesh = plsc.VectorSubcoreMesh(core_axis_name="c", subcore_axis_name="s")


@functools.partial(
    pl.kernel,
    out_type=jax.ShapeDtypeStruct((NC, NPAD), jnp.float32),
    mesh=_mesh,
    scratch_types=[
        pltpu.VMEM((NB_E, EB), jnp.int32),
        pltpu.VMEM((EB,), jnp.float32),
        pltpu.VMEM_SHARED((NPAD,), jnp.float32),
    ],
)
def _deg_kernel(dstb_hbm, ones_hbm, zero_hbm, out_hbm, idx_v, ones_v, acc):
    c = lax.axis_index("c")
    s = lax.axis_index("s")
    tid = c * NS + s
    pltpu.sync_copy(zero_hbm, acc.at[pl.ds(s * ROWS_T, ROWS_T)])
    pltpu.sync_copy(ones_hbm, ones_v)
    pltpu.sync_copy(dstb_hbm.at[tid], idx_v)
    plsc.subcore_barrier()

    def body(j, carry):
        pltpu.sync_copy(ones_v, acc.at[idx_v.at[j]], add=True)
        return carry

    lax.fori_loop(0, NB_E, body, 0)
    plsc.subcore_barrier()
    pltpu.sync_copy(acc.at[pl.ds(s * ROWS_T, ROWS_T)],
                    out_hbm.at[c, pl.ds(s * ROWS_T, ROWS_T)])


@functools.partial(
    pl.kernel,
    out_type=jax.ShapeDtypeStruct((NC, NPAD, H), jnp.float32),
    mesh=_mesh,
    scratch_types=[
        pltpu.VMEM((NB_E, EB), jnp.int32),
        pltpu.VMEM((NB_E, EB), jnp.int32),
        pltpu.VMEM((EB, H), jnp.float32),
        pltpu.VMEM_SHARED((NPAD, H), jnp.float32),
        pltpu.SemaphoreType.DMA,
    ],
)
def _agg_kernel(h_hbm, srcb_hbm, dstb_hbm, zero_hbm, out_hbm,
                src_v, dst_v, rows_v, acc, sem):
    c = lax.axis_index("c")
    s = lax.axis_index("s")
    tid = c * NS + s
    pltpu.sync_copy(zero_hbm, acc.at[pl.ds(s * ROWS_T, ROWS_T)])
    pltpu.sync_copy(srcb_hbm.at[tid], src_v)
    pltpu.sync_copy(dstb_hbm.at[tid], dst_v)
    plsc.subcore_barrier()

    def body(j, carry):
        pltpu.async_copy(h_hbm.at[src_v.at[j]], rows_v, sem).wait()
        pltpu.sync_copy(rows_v, acc.at[dst_v.at[j]], add=True)
        return carry

    lax.fori_loop(0, NB_E, body, 0)
    plsc.subcore_barrier()
    pltpu.sync_copy(acc.at[pl.ds(s * ROWS_T, ROWS_T)],
                    out_hbm.at[c, pl.ds(s * ROWS_T, ROWS_T)])


@functools.partial(
    pl.kernel,
    out_type=jax.ShapeDtypeStruct((2 * P, H), jnp.float32),
    mesh=_mesh,
    scratch_types=[
        pltpu.VMEM((NB_P, EB), jnp.int32),
        pltpu.VMEM((EB, H), jnp.float32),
        pltpu.SemaphoreType.DMA,
    ],
)
def _pair_gather_kernel(z_hbm, pairs_hbm, out_hbm, idx_v, rows_v, sem):
    c = lax.axis_index("c")
    s = lax.axis_index("s")
    tid = c * NS + s
    pltpu.sync_copy(pairs_hbm.at[tid], idx_v)

    def body(j, carry):
        pltpu.async_copy(z_hbm.at[idx_v.at[j]], rows_v, sem).wait()
        pltpu.sync_copy(rows_v, out_hbm.at[pl.ds(tid * PB + j * EB, EB)])
        return carry

    lax.fori_loop(0, NB_P, body, 0)


def _dinv_body(d0_ref, d1_ref, o_ref):
    o_ref[...] = lax.rsqrt(d0_ref[...] + d1_ref[...] + 1.0)


_dinv_tc = pl.pallas_call(
    _dinv_body, out_shape=jax.ShapeDtypeStruct((N, 1), jnp.float32))


def _mm_scale_body(x_ref, w_ref, dinv_ref, o_ref):
    o_ref[...] = jnp.dot(x_ref[...], w_ref[...],
                         preferred_element_type=jnp.float32) * dinv_ref[...]


_mm_scale_tc = pl.pallas_call(
    _mm_scale_body, out_shape=jax.ShapeDtypeStruct((N, H), jnp.float32))


def _fin_mm_body(a0_ref, a1_ref, hp_ref, dinv_ref, b_ref, w_ref, o_ref):
    z = dinv_ref[...] * (a0_ref[...] + a1_ref[...] + hp_ref[...]) + b_ref[...]
    z = jnp.maximum(z, 0.0)
    o_ref[...] = jnp.dot(z, w_ref[...],
                         preferred_element_type=jnp.float32) * dinv_ref[...]


_fin_mm_tc = pl.pallas_call(
    _fin_mm_body, out_shape=jax.ShapeDtypeStruct((N, H), jnp.float32))


def _fin2_body(a0_ref, a1_ref, hp_ref, dinv_ref, b_ref, o_ref):
    o_ref[...] = (dinv_ref[...] * (a0_ref[...] + a1_ref[...] + hp_ref[...])
                  + b_ref[...])


_fin2_tc = pl.pallas_call(
    _fin2_body, out_shape=jax.ShapeDtypeStruct((N, H), jnp.float32))


LB = 8192


def _link_body(zs_ref, zd_ref, wa_ref, wb_ref, b1_ref, w2_ref, b2_ref, o_ref):
    t = jnp.dot(zs_ref[...], wa_ref[...], preferred_element_type=jnp.float32)
    t = t + jnp.dot(zd_ref[...], wb_ref[...], preferred_element_type=jnp.float32)
    hr = jnp.maximum(t + b1_ref[...], 0.0)
    logit = jnp.sum(hr * w2_ref[...], axis=1, keepdims=True) + b2_ref[...]
    o_ref[...] = 1.0 / (1.0 + jnp.exp(-logit))


_link_tc = pl.pallas_call(
    _link_body,
    grid=(P // LB,),
    in_specs=[
        pl.BlockSpec((LB, H), lambda i: (i, 0)),
        pl.BlockSpec((LB, H), lambda i: (i, 0)),
        pl.BlockSpec((H, H), lambda i: (0, 0)),
        pl.BlockSpec((H, H), lambda i: (0, 0)),
        pl.BlockSpec((1, H), lambda i: (0, 0)),
        pl.BlockSpec((1, H), lambda i: (0, 0)),
        pl.BlockSpec((1, 1), lambda i: (0, 0)),
    ],
    out_specs=pl.BlockSpec((LB, 1), lambda i: (i, 0)),
    out_shape=jax.ShapeDtypeStruct((P, 1), jnp.float32),
)


def kernel(x, edge_index_train, edge_pairs, W1, b1, W2, b2,
           lpW1, lpb1, lpW2, lpb2):
    src = edge_index_train[0].astype(jnp.int32)
    dst = edge_index_train[1].astype(jnp.int32)
    pad = EPAD - E
    srcb = jnp.concatenate([src, jnp.zeros((pad,), jnp.int32)])
    dstb = jnp.concatenate([dst, jnp.full((pad,), N, jnp.int32)])
    srcb = srcb.reshape(NT, NB_E, EB)
    dstb = dstb.reshape(NT, NB_E, EB)
    pairsb = edge_pairs.astype(jnp.int32).reshape(NT, NB_P, EB)

    ones_eb = jnp.ones((EB,), jnp.float32)
    zero_1d = jnp.zeros((ROWS_T,), jnp.float32)
    zero_2d = jnp.zeros((ROWS_T, H), jnp.float32)

    degp = _deg_kernel(dstb, ones_eb, zero_1d)
    dinv = _dinv_tc(degp[0, :N].reshape(N, 1), degp[1, :N].reshape(N, 1))

    h1p = _mm_scale_tc(x, W1, dinv)
    agg1 = _agg_kernel(h1p, srcb, dstb, zero_2d)
    h2p = _fin_mm_tc(agg1[0, :N], agg1[1, :N], h1p, dinv,
                     b1.reshape(1, H), W2)
    agg2 = _agg_kernel(h2p, srcb, dstb, zero_2d)
    z2 = _fin2_tc(agg2[0, :N], agg2[1, :N], h2p, dinv, b2.reshape(1, H))

    rows = _pair_gather_kernel(z2, pairsb)
    out = _link_tc(rows[:P], rows[P:], lpW1[:H], lpW1[H:],
                   lpb1.reshape(1, H), lpW2.reshape(1, H),
                   lpb2.reshape(1, 1))
    return out.reshape(P)


# speedup vs baseline: 7.5476x; 1.1141x over previous
import functools

import jax
import jax.numpy as jnp
from jax import lax
from jax.experimental import pallas as pl
from jax.experimental.pallas import tpu as pltpu
from jax.experimental.pallas import tpu_sc as plsc

N = 10000
D = 128
H = 128
E = 320000
P = 65536

NC = 2
NS = 16
NT = NC * NS
NPAD = 10240
ROWS_T = NPAD // NS
EB = 128
EPT = 10240
NB_E = EPT // EB
EPAD = NT * EPT
PB = (2 * P) // NT
NB_P = PB // EB

_mesh = plsc.VectorSubcoreMesh(core_axis_name="c", subcore_axis_name="s")


@functools.partial(
    pl.kernel,
    out_type=jax.ShapeDtypeStruct((NC, NPAD), jnp.float32),
    mesh=_mesh,
    scratch_types=[
        pltpu.VMEM((NB_E, EB), jnp.int32),
        pltpu.VMEM((EB,), jnp.float32),
        pltpu.VMEM_SHARED((NPAD,), jnp.float32),
    ],
)
def _deg_kernel(dstb_hbm, ones_hbm, zero_hbm, out_hbm, idx_v, ones_v, acc):
    c = lax.axis_index("c")
    s = lax.axis_index("s")
    tid = c * NS + s
    pltpu.sync_copy(zero_hbm, acc.at[pl.ds(s * ROWS_T, ROWS_T)])
    pltpu.sync_copy(ones_hbm, ones_v)
    pltpu.sync_copy(dstb_hbm.at[tid], idx_v)
    plsc.subcore_barrier()

    def body(j, carry):
        pltpu.sync_copy(ones_v, acc.at[idx_v.at[j]], add=True)
        return carry

    lax.fori_loop(0, NB_E, body, 0)
    plsc.subcore_barrier()
    pltpu.sync_copy(acc.at[pl.ds(s * ROWS_T, ROWS_T)],
                    out_hbm.at[c, pl.ds(s * ROWS_T, ROWS_T)])


@functools.partial(
    pl.kernel,
    out_type=jax.ShapeDtypeStruct((NC, NPAD, H), jnp.float32),
    mesh=_mesh,
    scratch_types=[
        pltpu.VMEM((NB_E // 2, EB), jnp.int32),
        pltpu.VMEM((NB_E // 2, EB), jnp.int32),
        pltpu.VMEM((EB, H), jnp.float32),
        pltpu.VMEM((EB, H), jnp.float32),
        pltpu.VMEM_SHARED((NPAD, H), jnp.float32),
        pltpu.SemaphoreType.DMA,
        pltpu.SemaphoreType.DMA,
    ],
)
def _agg_kernel(h_hbm, srcb_hbm, dstb_hbm, zero_hbm, out_hbm,
                src_v, dst_v, rows0, rows1, acc, sem0, sem1):
    c = lax.axis_index("c")
    s = lax.axis_index("s")
    tid = c * NS + s
    nph = NB_E // 2
    pltpu.sync_copy(zero_hbm, acc.at[pl.ds(s * ROWS_T, ROWS_T)])
    plsc.subcore_barrier()

    for p in range(2):
        pltpu.sync_copy(srcb_hbm.at[tid, pl.ds(p * nph, nph)], src_v)
        pltpu.sync_copy(dstb_hbm.at[tid, pl.ds(p * nph, nph)], dst_v)
        pltpu.async_copy(h_hbm.at[src_v.at[0]], rows0, sem0)
        pltpu.async_copy(h_hbm.at[src_v.at[1]], rows1, sem1)

        def body(jj, carry):
            j0 = jj * 2
            pltpu.make_async_copy(h_hbm.at[src_v.at[j0]], rows0, sem0).wait()
            pltpu.sync_copy(rows0, acc.at[dst_v.at[j0]], add=True)
            pltpu.async_copy(h_hbm.at[src_v.at[j0 + 2]], rows0, sem0)
            pltpu.make_async_copy(h_hbm.at[src_v.at[j0]], rows1, sem1).wait()
            pltpu.sync_copy(rows1, acc.at[dst_v.at[j0 + 1]], add=True)
            pltpu.async_copy(h_hbm.at[src_v.at[j0 + 3]], rows1, sem1)
            return carry

        lax.fori_loop(0, nph // 2 - 1, body, 0)
        pltpu.make_async_copy(h_hbm.at[src_v.at[0]], rows0, sem0).wait()
        pltpu.sync_copy(rows0, acc.at[dst_v.at[nph - 2]], add=True)
        pltpu.make_async_copy(h_hbm.at[src_v.at[0]], rows1, sem1).wait()
        pltpu.sync_copy(rows1, acc.at[dst_v.at[nph - 1]], add=True)
    plsc.subcore_barrier()
    pltpu.sync_copy(acc.at[pl.ds(s * ROWS_T, ROWS_T)],
                    out_hbm.at[c, pl.ds(s * ROWS_T, ROWS_T)])


@functools.partial(
    pl.kernel,
    out_type=jax.ShapeDtypeStruct((2 * P, H), jnp.float32),
    mesh=_mesh,
    scratch_types=[
        pltpu.VMEM((NB_P, EB), jnp.int32),
        pltpu.VMEM((EB, H), jnp.float32),
        pltpu.VMEM((EB, H), jnp.float32),
        pltpu.SemaphoreType.DMA,
        pltpu.SemaphoreType.DMA,
    ],
)
def _pair_gather_kernel(z_hbm, pairs_hbm, out_hbm, idx_v, rows0, rows1,
                        sem0, sem1):
    c = lax.axis_index("c")
    s = lax.axis_index("s")
    tid = c * NS + s
    pltpu.sync_copy(pairs_hbm.at[tid], idx_v)
    base = tid * PB

    pltpu.async_copy(z_hbm.at[idx_v.at[0]], rows0, sem0)
    pltpu.async_copy(z_hbm.at[idx_v.at[1]], rows1, sem1)

    def body(jj, carry):
        j0 = jj * 2
        pltpu.make_async_copy(z_hbm.at[idx_v.at[j0]], rows0, sem0).wait()
        pltpu.sync_copy(rows0, out_hbm.at[pl.ds(base + j0 * EB, EB)])
        pltpu.async_copy(z_hbm.at[idx_v.at[j0 + 2]], rows0, sem0)
        pltpu.make_async_copy(z_hbm.at[idx_v.at[j0]], rows1, sem1).wait()
        pltpu.sync_copy(rows1, out_hbm.at[pl.ds(base + (j0 + 1) * EB, EB)])
        pltpu.async_copy(z_hbm.at[idx_v.at[j0 + 3]], rows1, sem1)
        return carry

    lax.fori_loop(0, NB_P // 2 - 1, body, 0)
    pltpu.make_async_copy(z_hbm.at[idx_v.at[0]], rows0, sem0).wait()
    pltpu.sync_copy(rows0, out_hbm.at[pl.ds(base + (NB_P - 2) * EB, EB)])
    pltpu.make_async_copy(z_hbm.at[idx_v.at[0]], rows1, sem1).wait()
    pltpu.sync_copy(rows1, out_hbm.at[pl.ds(base + (NB_P - 1) * EB, EB)])


def _dinv_body(d0_ref, d1_ref, o_ref):
    o_ref[...] = lax.rsqrt(d0_ref[...] + d1_ref[...] + 1.0)


_dinv_tc = pl.pallas_call(
    _dinv_body, out_shape=jax.ShapeDtypeStruct((N, 1), jnp.float32))


def _mm_scale_body(x_ref, w_ref, dinv_ref, o_ref):
    o_ref[...] = jnp.dot(x_ref[...], w_ref[...],
                         preferred_element_type=jnp.float32) * dinv_ref[...]


_mm_scale_tc = pl.pallas_call(
    _mm_scale_body, out_shape=jax.ShapeDtypeStruct((N, H), jnp.float32))


def _fin_mm_body(a0_ref, a1_ref, hp_ref, dinv_ref, b_ref, w_ref, o_ref):
    z = dinv_ref[...] * (a0_ref[...] + a1_ref[...] + hp_ref[...]) + b_ref[...]
    z = jnp.maximum(z, 0.0)
    o_ref[...] = jnp.dot(z, w_ref[...],
                         preferred_element_type=jnp.float32) * dinv_ref[...]


_fin_mm_tc = pl.pallas_call(
    _fin_mm_body, out_shape=jax.ShapeDtypeStruct((N, H), jnp.float32))


def _fin2_body(a0_ref, a1_ref, hp_ref, dinv_ref, b_ref, o_ref):
    o_ref[...] = (dinv_ref[...] * (a0_ref[...] + a1_ref[...] + hp_ref[...])
                  + b_ref[...])


_fin2_tc = pl.pallas_call(
    _fin2_body, out_shape=jax.ShapeDtypeStruct((N, H), jnp.float32))


LB = 8192


def _link_body(zs_ref, zd_ref, wa_ref, wb_ref, b1_ref, w2_ref, b2_ref, o_ref):
    t = jnp.dot(zs_ref[...], wa_ref[...], preferred_element_type=jnp.float32)
    t = t + jnp.dot(zd_ref[...], wb_ref[...], preferred_element_type=jnp.float32)
    hr = jnp.maximum(t + b1_ref[...], 0.0)
    logit = jnp.sum(hr * w2_ref[...], axis=1, keepdims=True) + b2_ref[...]
    o_ref[...] = 1.0 / (1.0 + jnp.exp(-logit))


_link_tc = pl.pallas_call(
    _link_body,
    grid=(P // LB,),
    in_specs=[
        pl.BlockSpec((LB, H), lambda i: (i, 0)),
        pl.BlockSpec((LB, H), lambda i: (i, 0)),
        pl.BlockSpec((H, H), lambda i: (0, 0)),
        pl.BlockSpec((H, H), lambda i: (0, 0)),
        pl.BlockSpec((1, H), lambda i: (0, 0)),
        pl.BlockSpec((1, H), lambda i: (0, 0)),
        pl.BlockSpec((1, 1), lambda i: (0, 0)),
    ],
    out_specs=pl.BlockSpec((LB, 1), lambda i: (i, 0)),
    out_shape=jax.ShapeDtypeStruct((P, 1), jnp.float32),
)


def kernel(x, edge_index_train, edge_pairs, W1, b1, W2, b2,
           lpW1, lpb1, lpW2, lpb2):
    src = edge_index_train[0].astype(jnp.int32)
    dst = edge_index_train[1].astype(jnp.int32)
    pad = EPAD - E
    srcb = jnp.concatenate([src, jnp.zeros((pad,), jnp.int32)])
    pad_dst = N + (jnp.arange(pad, dtype=jnp.int32) % (NPAD - N))
    dstb = jnp.concatenate([dst, pad_dst])
    srcb = srcb.reshape(NT, NB_E, EB)
    dstb = dstb.reshape(NT, NB_E, EB)
    pairsb = edge_pairs.astype(jnp.int32).reshape(NT, NB_P, EB)

    ones_eb = jnp.ones((EB,), jnp.float32)
    zero_1d = jnp.zeros((ROWS_T,), jnp.float32)
    zero_2d = jnp.zeros((ROWS_T, H), jnp.float32)

    degp = _deg_kernel(dstb, ones_eb, zero_1d)
    dinv = _dinv_tc(degp[0, :N].reshape(N, 1), degp[1, :N].reshape(N, 1))

    h1p = _mm_scale_tc(x, W1, dinv)
    agg1 = _agg_kernel(h1p, srcb, dstb, zero_2d)
    h2p = _fin_mm_tc(agg1[0, :N], agg1[1, :N], h1p, dinv,
                     b1.reshape(1, H), W2)
    agg2 = _agg_kernel(h2p, srcb, dstb, zero_2d)
    z2 = _fin2_tc(agg2[0, :N], agg2[1, :N], h2p, dinv, b2.reshape(1, H))

    rows = _pair_gather_kernel(z2, pairsb)
    out = _link_tc(rows[:P], rows[P:], lpW1[:H], lpW1[H:],
                   lpb1.reshape(1, H), lpW2.reshape(1, H),
                   lpb2.reshape(1, 1))
    return out.reshape(P)


# speedup vs baseline: 7.5510x; 1.0005x over previous
import functools

import jax
import jax.numpy as jnp
from jax import lax
from jax.experimental import pallas as pl
from jax.experimental.pallas import tpu as pltpu
from jax.experimental.pallas import tpu_sc as plsc

N = 10000
D = 128
H = 128
E = 320000
P = 65536

NC = 2
NS = 16
NT = NC * NS
NPAD = 10240
ROWS_T = NPAD // NS
EB = 128
EPT = 10240
NB_E = EPT // EB
EPAD = NT * EPT
PB = (2 * P) // NT
NB_P = PB // EB

_mesh = plsc.VectorSubcoreMesh(core_axis_name="c", subcore_axis_name="s")


@functools.partial(
    pl.kernel,
    out_type=jax.ShapeDtypeStruct((NC, NPAD), jnp.float32),
    mesh=_mesh,
    scratch_types=[
        pltpu.VMEM((NB_E, EB), jnp.int32),
        pltpu.VMEM((EB,), jnp.float32),
        pltpu.VMEM_SHARED((NPAD,), jnp.float32),
    ],
)
def _deg_kernel(dstb_hbm, ones_hbm, zero_hbm, out_hbm, idx_v, ones_v, acc):
    c = lax.axis_index("c")
    s = lax.axis_index("s")
    tid = c * NS + s
    pltpu.sync_copy(zero_hbm, acc.at[pl.ds(s * ROWS_T, ROWS_T)])
    pltpu.sync_copy(ones_hbm, ones_v)
    pltpu.sync_copy(dstb_hbm.at[tid], idx_v)
    plsc.subcore_barrier()

    def body(j, carry):
        pltpu.sync_copy(ones_v, acc.at[idx_v.at[j]], add=True)
        return carry

    lax.fori_loop(0, NB_E, body, 0)
    plsc.subcore_barrier()
    pltpu.sync_copy(acc.at[pl.ds(s * ROWS_T, ROWS_T)],
                    out_hbm.at[c, pl.ds(s * ROWS_T, ROWS_T)])


@functools.partial(
    pl.kernel,
    out_type=jax.ShapeDtypeStruct((NC, NPAD, H), jnp.float32),
    mesh=_mesh,
    scratch_types=[
        pltpu.VMEM((NB_E // 2, EB), jnp.int32),
        pltpu.VMEM((NB_E // 2, EB), jnp.int32),
        pltpu.VMEM((EB, H), jnp.float32),
        pltpu.VMEM((EB, H), jnp.float32),
        pltpu.VMEM_SHARED((NPAD, H), jnp.float32),
        pltpu.SemaphoreType.DMA,
        pltpu.SemaphoreType.DMA,
    ],
)
def _agg_kernel(h_hbm, srcb_hbm, dstb_hbm, zero_hbm, out_hbm,
                src_v, dst_v, rows0, rows1, acc, sem0, sem1):
    c = lax.axis_index("c")
    s = lax.axis_index("s")
    tid = s * NC + c
    nph = NB_E // 2
    pltpu.sync_copy(zero_hbm, acc.at[pl.ds(s * ROWS_T, ROWS_T)])
    plsc.subcore_barrier()

    for p in range(2):
        pltpu.sync_copy(srcb_hbm.at[tid, pl.ds(p * nph, nph)], src_v)
        pltpu.sync_copy(dstb_hbm.at[tid, pl.ds(p * nph, nph)], dst_v)
        pltpu.async_copy(h_hbm.at[src_v.at[0]], rows0, sem0)
        pltpu.async_copy(h_hbm.at[src_v.at[1]], rows1, sem1)

        def body(jj, carry):
            j0 = jj * 2
            pltpu.make_async_copy(h_hbm.at[src_v.at[j0]], rows0, sem0).wait()
            pltpu.sync_copy(rows0, acc.at[dst_v.at[j0]], add=True)
            pltpu.async_copy(h_hbm.at[src_v.at[j0 + 2]], rows0, sem0)
            pltpu.make_async_copy(h_hbm.at[src_v.at[j0]], rows1, sem1).wait()
            pltpu.sync_copy(rows1, acc.at[dst_v.at[j0 + 1]], add=True)
            pltpu.async_copy(h_hbm.at[src_v.at[j0 + 3]], rows1, sem1)
            return carry

        lax.fori_loop(0, nph // 2 - 1, body, 0)
        pltpu.make_async_copy(h_hbm.at[src_v.at[0]], rows0, sem0).wait()
        pltpu.sync_copy(rows0, acc.at[dst_v.at[nph - 2]], add=True)
        pltpu.make_async_copy(h_hbm.at[src_v.at[0]], rows1, sem1).wait()
        pltpu.sync_copy(rows1, acc.at[dst_v.at[nph - 1]], add=True)
    plsc.subcore_barrier()
    pltpu.sync_copy(acc.at[pl.ds(s * ROWS_T, ROWS_T)],
                    out_hbm.at[c, pl.ds(s * ROWS_T, ROWS_T)])


@functools.partial(
    pl.kernel,
    out_type=jax.ShapeDtypeStruct((2 * P, H), jnp.float32),
    mesh=_mesh,
    scratch_types=[
        pltpu.VMEM((NB_P, EB), jnp.int32),
        pltpu.VMEM((EB, H), jnp.float32),
        pltpu.VMEM((EB, H), jnp.float32),
        pltpu.SemaphoreType.DMA,
        pltpu.SemaphoreType.DMA,
    ],
)
def _pair_gather_kernel(z_hbm, pairs_hbm, out_hbm, idx_v, rows0, rows1,
                        sem0, sem1):
    c = lax.axis_index("c")
    s = lax.axis_index("s")
    tid = c * NS + s
    pltpu.sync_copy(pairs_hbm.at[tid], idx_v)
    base = tid * PB

    pltpu.async_copy(z_hbm.at[idx_v.at[0]], rows0, sem0)
    pltpu.async_copy(z_hbm.at[idx_v.at[1]], rows1, sem1)

    def body(jj, carry):
        j0 = jj * 2
        pltpu.make_async_copy(z_hbm.at[idx_v.at[j0]], rows0, sem0).wait()
        pltpu.sync_copy(rows0, out_hbm.at[pl.ds(base + j0 * EB, EB)])
        pltpu.async_copy(z_hbm.at[idx_v.at[j0 + 2]], rows0, sem0)
        pltpu.make_async_copy(z_hbm.at[idx_v.at[j0]], rows1, sem1).wait()
        pltpu.sync_copy(rows1, out_hbm.at[pl.ds(base + (j0 + 1) * EB, EB)])
        pltpu.async_copy(z_hbm.at[idx_v.at[j0 + 3]], rows1, sem1)
        return carry

    lax.fori_loop(0, NB_P // 2 - 1, body, 0)
    pltpu.make_async_copy(z_hbm.at[idx_v.at[0]], rows0, sem0).wait()
    pltpu.sync_copy(rows0, out_hbm.at[pl.ds(base + (NB_P - 2) * EB, EB)])
    pltpu.make_async_copy(z_hbm.at[idx_v.at[0]], rows1, sem1).wait()
    pltpu.sync_copy(rows1, out_hbm.at[pl.ds(base + (NB_P - 1) * EB, EB)])


def _dinv_body(d0_ref, d1_ref, o_ref):
    o_ref[...] = lax.rsqrt(d0_ref[...] + d1_ref[...] + 1.0)


_dinv_tc = pl.pallas_call(
    _dinv_body, out_shape=jax.ShapeDtypeStruct((N, 1), jnp.float32))


def _mm_scale_body(x_ref, w_ref, dinv_ref, o_ref):
    o_ref[...] = jnp.dot(x_ref[...], w_ref[...],
                         preferred_element_type=jnp.float32) * dinv_ref[...]


_mm_scale_tc = pl.pallas_call(
    _mm_scale_body, out_shape=jax.ShapeDtypeStruct((N, H), jnp.float32))


def _fin_mm_body(a0_ref, a1_ref, hp_ref, dinv_ref, b_ref, w_ref, o_ref):
    z = dinv_ref[...] * (a0_ref[...] + a1_ref[...] + hp_ref[...]) + b_ref[...]
    z = jnp.maximum(z, 0.0)
    o_ref[...] = jnp.dot(z, w_ref[...],
                         preferred_element_type=jnp.float32) * dinv_ref[...]


_fin_mm_tc = pl.pallas_call(
    _fin_mm_body, out_shape=jax.ShapeDtypeStruct((N, H), jnp.float32))


def _fin2_body(a0_ref, a1_ref, hp_ref, dinv_ref, b_ref, o_ref):
    o_ref[...] = (dinv_ref[...] * (a0_ref[...] + a1_ref[...] + hp_ref[...])
                  + b_ref[...])


_fin2_tc = pl.pallas_call(
    _fin2_body, out_shape=jax.ShapeDtypeStruct((N, H), jnp.float32))


LB = 8192


def _link_body(zs_ref, zd_ref, wa_ref, wb_ref, b1_ref, w2_ref, b2_ref, o_ref):
    t = jnp.dot(zs_ref[...], wa_ref[...], preferred_element_type=jnp.float32)
    t = t + jnp.dot(zd_ref[...], wb_ref[...], preferred_element_type=jnp.float32)
    hr = jnp.maximum(t + b1_ref[...], 0.0)
    logit = jnp.sum(hr * w2_ref[...], axis=1, keepdims=True) + b2_ref[...]
    o_ref[...] = 1.0 / (1.0 + jnp.exp(-logit))


_link_tc = pl.pallas_call(
    _link_body,
    grid=(P // LB,),
    in_specs=[
        pl.BlockSpec((LB, H), lambda i: (i, 0)),
        pl.BlockSpec((LB, H), lambda i: (i, 0)),
        pl.BlockSpec((H, H), lambda i: (0, 0)),
        pl.BlockSpec((H, H), lambda i: (0, 0)),
        pl.BlockSpec((1, H), lambda i: (0, 0)),
        pl.BlockSpec((1, H), lambda i: (0, 0)),
        pl.BlockSpec((1, 1), lambda i: (0, 0)),
    ],
    out_specs=pl.BlockSpec((LB, 1), lambda i: (i, 0)),
    out_shape=jax.ShapeDtypeStruct((P, 1), jnp.float32),
)


def kernel(x, edge_index_train, edge_pairs, W1, b1, W2, b2,
           lpW1, lpb1, lpW2, lpb2):
    src = edge_index_train[0].astype(jnp.int32)
    dst = edge_index_train[1].astype(jnp.int32)
    pad = EPAD - E
    srcb = jnp.concatenate([src, jnp.zeros((pad,), jnp.int32)])
    pad_dst = N + (jnp.arange(pad, dtype=jnp.int32) % (NPAD - N))
    dstb = jnp.concatenate([dst, pad_dst])
    srcb = srcb.reshape(NT, NB_E, EB)
    dstb = dstb.reshape(NT, NB_E, EB)
    pairsb = edge_pairs.astype(jnp.int32).reshape(NT, NB_P, EB)

    ones_eb = jnp.ones((EB,), jnp.float32)
    zero_1d = jnp.zeros((ROWS_T,), jnp.float32)
    zero_2d = jnp.zeros((ROWS_T, H), jnp.float32)

    degp = _deg_kernel(dstb, ones_eb, zero_1d)
    dinv = _dinv_tc(degp[0, :N].reshape(N, 1), degp[1, :N].reshape(N, 1))

    h1p = _mm_scale_tc(x, W1, dinv)
    agg1 = _agg_kernel(h1p, srcb, dstb, zero_2d)
    h2p = _fin_mm_tc(agg1[0, :N], agg1[1, :N], h1p, dinv,
                     b1.reshape(1, H), W2)
    agg2 = _agg_kernel(h2p, srcb, dstb, zero_2d)
    z2 = _fin2_tc(agg2[0, :N], agg2[1, :N], h2p, dinv, b2.reshape(1, H))

    rows = _pair_gather_kernel(z2, pairsb)
    out = _link_tc(rows[:P], rows[P:], lpW1[:H], lpW1[H:],
                   lpb1.reshape(1, H), lpW2.reshape(1, H),
                   lpb2.reshape(1, 1))
    return out.reshape(P)


# speedup vs baseline: 21.8887x; 2.8988x over previous
import functools

import jax
import jax.numpy as jnp
from jax import lax
from jax.experimental import pallas as pl
from jax.experimental.pallas import tpu as pltpu
from jax.experimental.pallas import tpu_sc as plsc

N = 10000
D = 128
H = 128
E = 320000
P = 65536

NC = 2
NS = 16
NT = NC * NS
NPAD = 10240
ROWS_T = NPAD // NS
EB = 128
EPT = 10240
NB_E = EPT // EB
EPAD = NT * EPT
PB = (2 * P) // NT
NB_P = PB // EB

_mesh = plsc.VectorSubcoreMesh(core_axis_name="c", subcore_axis_name="s")


@functools.partial(
    pl.kernel,
    out_type=jax.ShapeDtypeStruct((NC, NPAD), jnp.float32),
    mesh=_mesh,
    scratch_types=[
        pltpu.VMEM((NB_E, EB), jnp.int32),
        pltpu.VMEM((EB,), jnp.float32),
        pltpu.VMEM_SHARED((NPAD,), jnp.float32),
    ],
)
def _deg_kernel(dstb_hbm, ones_hbm, zero_hbm, out_hbm, idx_v, ones_v, acc):
    c = lax.axis_index("c")
    s = lax.axis_index("s")
    tid = c * NS + s
    pltpu.sync_copy(zero_hbm, acc.at[pl.ds(s * ROWS_T, ROWS_T)])
    pltpu.sync_copy(ones_hbm, ones_v)
    pltpu.sync_copy(dstb_hbm.at[tid], idx_v)
    plsc.subcore_barrier()

    def body(j, carry):
        pltpu.sync_copy(ones_v, acc.at[idx_v.at[j]], add=True)
        return carry

    lax.fori_loop(0, NB_E, body, 0)
    plsc.subcore_barrier()
    pltpu.sync_copy(acc.at[pl.ds(s * ROWS_T, ROWS_T)],
                    out_hbm.at[c, pl.ds(s * ROWS_T, ROWS_T)])


@functools.partial(
    pl.kernel,
    out_type=jax.ShapeDtypeStruct((NC, NPAD, H), jnp.float32),
    mesh=_mesh,
    scratch_types=[
        pltpu.VMEM((NB_E // 2, EB), jnp.int32),
        pltpu.VMEM((NB_E // 2, EB), jnp.int32),
        pltpu.VMEM((EB, H), jnp.float32),
        pltpu.VMEM((EB, H), jnp.float32),
        pltpu.VMEM_SHARED((NPAD, H), jnp.float32),
        pltpu.SemaphoreType.DMA,
        pltpu.SemaphoreType.DMA,
    ],
)
def _agg_kernel(h_hbm, srcb_hbm, dstb_hbm, zero_hbm, out_hbm,
                src_v, dst_v, rows0, rows1, acc, sem0, sem1):
    c = lax.axis_index("c")
    s = lax.axis_index("s")
    tid = s * NC + c
    nph = NB_E // 2
    pltpu.sync_copy(zero_hbm, acc.at[pl.ds(s * ROWS_T, ROWS_T)])
    plsc.subcore_barrier()

    for p in range(2):
        pltpu.sync_copy(srcb_hbm.at[tid, pl.ds(p * nph, nph)], src_v)
        pltpu.sync_copy(dstb_hbm.at[tid, pl.ds(p * nph, nph)], dst_v)
        pltpu.async_copy(h_hbm.at[src_v.at[0]], rows0, sem0)
        pltpu.async_copy(h_hbm.at[src_v.at[1]], rows1, sem1)

        def body(jj, carry):
            j0 = jj * 2
            pltpu.make_async_copy(h_hbm.at[src_v.at[j0]], rows0, sem0).wait()
            pltpu.sync_copy(rows0, acc.at[dst_v.at[j0]], add=True)
            pltpu.async_copy(h_hbm.at[src_v.at[j0 + 2]], rows0, sem0)
            pltpu.make_async_copy(h_hbm.at[src_v.at[j0]], rows1, sem1).wait()
            pltpu.sync_copy(rows1, acc.at[dst_v.at[j0 + 1]], add=True)
            pltpu.async_copy(h_hbm.at[src_v.at[j0 + 3]], rows1, sem1)
            return carry

        lax.fori_loop(0, nph // 2 - 1, body, 0)
        pltpu.make_async_copy(h_hbm.at[src_v.at[0]], rows0, sem0).wait()
        pltpu.sync_copy(rows0, acc.at[dst_v.at[nph - 2]], add=True)
        pltpu.make_async_copy(h_hbm.at[src_v.at[0]], rows1, sem1).wait()
        pltpu.sync_copy(rows1, acc.at[dst_v.at[nph - 1]], add=True)
    plsc.subcore_barrier()
    pltpu.sync_copy(acc.at[pl.ds(s * ROWS_T, ROWS_T)],
                    out_hbm.at[c, pl.ds(s * ROWS_T, ROWS_T)])


@functools.partial(
    pl.kernel,
    out_type=jax.ShapeDtypeStruct((2 * P, H), jnp.float32),
    mesh=_mesh,
    scratch_types=[
        pltpu.VMEM((NB_P, EB), jnp.int32),
        pltpu.VMEM((EB, H), jnp.float32),
        pltpu.VMEM((EB, H), jnp.float32),
        pltpu.SemaphoreType.DMA,
        pltpu.SemaphoreType.DMA,
    ],
)
def _pair_gather_kernel(z_hbm, pairs_hbm, out_hbm, idx_v, rows0, rows1,
                        sem0, sem1):
    c = lax.axis_index("c")
    s = lax.axis_index("s")
    tid = c * NS + s
    pltpu.sync_copy(pairs_hbm.at[tid], idx_v)
    base = tid * PB

    pltpu.async_copy(z_hbm.at[idx_v.at[0]], rows0, sem0)
    pltpu.async_copy(z_hbm.at[idx_v.at[1]], rows1, sem1)

    def body(jj, carry):
        j0 = jj * 2
        pltpu.make_async_copy(z_hbm.at[idx_v.at[j0]], rows0, sem0).wait()
        pltpu.sync_copy(rows0, out_hbm.at[pl.ds(base + j0 * EB, EB)])
        pltpu.async_copy(z_hbm.at[idx_v.at[j0 + 2]], rows0, sem0)
        pltpu.make_async_copy(z_hbm.at[idx_v.at[j0]], rows1, sem1).wait()
        pltpu.sync_copy(rows1, out_hbm.at[pl.ds(base + (j0 + 1) * EB, EB)])
        pltpu.async_copy(z_hbm.at[idx_v.at[j0 + 3]], rows1, sem1)
        return carry

    lax.fori_loop(0, NB_P // 2 - 1, body, 0)
    pltpu.make_async_copy(z_hbm.at[idx_v.at[0]], rows0, sem0).wait()
    pltpu.sync_copy(rows0, out_hbm.at[pl.ds(base + (NB_P - 2) * EB, EB)])
    pltpu.make_async_copy(z_hbm.at[idx_v.at[0]], rows1, sem1).wait()
    pltpu.sync_copy(rows1, out_hbm.at[pl.ds(base + (NB_P - 1) * EB, EB)])


def _dinv_body(d0_ref, d1_ref, o_ref):
    o_ref[...] = lax.rsqrt(d0_ref[...] + d1_ref[...] + 1.0)


_dinv_tc = pl.pallas_call(
    _dinv_body, out_shape=jax.ShapeDtypeStruct((N, 1), jnp.float32))


def _mm_scale_body(x_ref, w_ref, dinv_ref, o_ref):
    o_ref[...] = jnp.dot(x_ref[...], w_ref[...],
                         preferred_element_type=jnp.float32) * dinv_ref[...]


_mm_scale_tc = pl.pallas_call(
    _mm_scale_body, out_shape=jax.ShapeDtypeStruct((N, H), jnp.float32))


def _fin_mm_body(a0_ref, a1_ref, hp_ref, dinv_ref, b_ref, w_ref, o_ref):
    z = dinv_ref[...] * (a0_ref[...] + a1_ref[...] + hp_ref[...]) + b_ref[...]
    z = jnp.maximum(z, 0.0)
    o_ref[...] = jnp.dot(z, w_ref[...],
                         preferred_element_type=jnp.float32) * dinv_ref[...]


_fin_mm_tc = pl.pallas_call(
    _fin_mm_body, out_shape=jax.ShapeDtypeStruct((N, H), jnp.float32))


def _fin2_body(a0_ref, a1_ref, hp_ref, dinv_ref, b_ref, o_ref):
    o_ref[...] = (dinv_ref[...] * (a0_ref[...] + a1_ref[...] + hp_ref[...])
                  + b_ref[...])


_fin2_tc = pl.pallas_call(
    _fin2_body, out_shape=jax.ShapeDtypeStruct((N, H), jnp.float32))


LB = 8192


def _link_body(zs_ref, zd_ref, wa_ref, wb_ref, b1_ref, w2_ref, b2_ref, o_ref):
    t = jnp.dot(zs_ref[...], wa_ref[...], preferred_element_type=jnp.float32)
    t = t + jnp.dot(zd_ref[...], wb_ref[...], preferred_element_type=jnp.float32)
    hr = jnp.maximum(t + b1_ref[...], 0.0)
    logit = jnp.sum(hr * w2_ref[...], axis=1, keepdims=True) + b2_ref[...]
    o_ref[...] = 1.0 / (1.0 + jnp.exp(-logit))


_link_tc = pl.pallas_call(
    _link_body,
    grid=(P // LB,),
    in_specs=[
        pl.BlockSpec((LB, H), lambda i: (i, 0)),
        pl.BlockSpec((LB, H), lambda i: (i, 0)),
        pl.BlockSpec((H, H), lambda i: (0, 0)),
        pl.BlockSpec((H, H), lambda i: (0, 0)),
        pl.BlockSpec((1, H), lambda i: (0, 0)),
        pl.BlockSpec((1, H), lambda i: (0, 0)),
        pl.BlockSpec((1, 1), lambda i: (0, 0)),
    ],
    out_specs=pl.BlockSpec((LB, 1), lambda i: (i, 0)),
    out_shape=jax.ShapeDtypeStruct((P, 1), jnp.float32),
)


def kernel(x, edge_index_train, edge_pairs, W1, b1, W2, b2,
           lpW1, lpb1, lpW2, lpb2):
    src = edge_index_train[0].astype(jnp.int32)
    dst = edge_index_train[1].astype(jnp.int32)
    pad = EPAD - E
    pad_src = jnp.arange(pad, dtype=jnp.int32) % N
    srcb = jnp.concatenate([src, pad_src])
    pad_dst = N + (jnp.arange(pad, dtype=jnp.int32) % (NPAD - N))
    dstb = jnp.concatenate([dst, pad_dst])
    srcb = srcb.reshape(NT, NB_E, EB)
    dstb = dstb.reshape(NT, NB_E, EB)
    pairsb = edge_pairs.astype(jnp.int32).reshape(NT, NB_P, EB)

    ones_eb = jnp.ones((EB,), jnp.float32)
    zero_1d = jnp.zeros((ROWS_T,), jnp.float32)
    zero_2d = jnp.zeros((ROWS_T, H), jnp.float32)

    degp = _deg_kernel(dstb, ones_eb, zero_1d)
    dinv = _dinv_tc(degp[0, :N].reshape(N, 1), degp[1, :N].reshape(N, 1))

    h1p = _mm_scale_tc(x, W1, dinv)
    agg1 = _agg_kernel(h1p, srcb, dstb, zero_2d)
    h2p = _fin_mm_tc(agg1[0, :N], agg1[1, :N], h1p, dinv,
                     b1.reshape(1, H), W2)
    agg2 = _agg_kernel(h2p, srcb, dstb, zero_2d)
    z2 = _fin2_tc(agg2[0, :N], agg2[1, :N], h2p, dinv, b2.reshape(1, H))

    rows = _pair_gather_kernel(z2, pairsb)
    out = _link_tc(rows[:P], rows[P:], lpW1[:H], lpW1[H:],
                   lpb1.reshape(1, H), lpW2.reshape(1, H),
                   lpb2.reshape(1, 1))
    return out.reshape(P)


# speedup vs baseline: 25.1391x; 1.1485x over previous
import functools

import jax
import jax.numpy as jnp
from jax import lax
from jax.experimental import pallas as pl
from jax.experimental.pallas import tpu as pltpu
from jax.experimental.pallas import tpu_sc as plsc

N = 10000
D = 128
H = 128
E = 320000
P = 65536

NC = 2
NS = 16
NT = NC * NS
NPAD = 10240
ROWS_T = NPAD // NS
EB = 128
EPT = 10240
NB_E = EPT // EB
EPAD = NT * EPT
PB = (2 * P) // NT
NB_P = PB // EB

_mesh = plsc.VectorSubcoreMesh(core_axis_name="c", subcore_axis_name="s")


@functools.partial(
    pl.kernel,
    out_type=jax.ShapeDtypeStruct((NC, NPAD), jnp.float32),
    mesh=_mesh,
    scratch_types=[
        pltpu.VMEM((NB_E, EB), jnp.int32),
        pltpu.VMEM((EB,), jnp.float32),
        pltpu.VMEM_SHARED((NPAD,), jnp.float32),
    ],
)
def _deg_kernel(dstb_hbm, ones_hbm, zero_hbm, out_hbm, idx_v, ones_v, acc):
    c = lax.axis_index("c")
    s = lax.axis_index("s")
    tid = c * NS + s
    pltpu.sync_copy(zero_hbm, acc.at[pl.ds(s * ROWS_T, ROWS_T)])
    pltpu.sync_copy(ones_hbm, ones_v)
    pltpu.sync_copy(dstb_hbm.at[tid], idx_v)
    plsc.subcore_barrier()

    def body(j, carry):
        pltpu.sync_copy(ones_v, acc.at[idx_v.at[j]], add=True)
        return carry

    lax.fori_loop(0, NB_E, body, 0)
    plsc.subcore_barrier()
    pltpu.sync_copy(acc.at[pl.ds(s * ROWS_T, ROWS_T)],
                    out_hbm.at[c, pl.ds(s * ROWS_T, ROWS_T)])


@functools.partial(
    pl.kernel,
    out_type=jax.ShapeDtypeStruct((NC, NPAD, H), jnp.float32),
    mesh=_mesh,
    scratch_types=[
        pltpu.VMEM((NB_E // 2, EB), jnp.int32),
        pltpu.VMEM((NB_E // 2, EB), jnp.int32),
        pltpu.VMEM((EB, H), jnp.float32),
        pltpu.VMEM((EB, H), jnp.float32),
        pltpu.VMEM_SHARED((NPAD, H), jnp.float32),
        pltpu.SemaphoreType.DMA,
        pltpu.SemaphoreType.DMA,
    ],
)
def _agg_kernel(h_hbm, srcb_hbm, dstb_hbm, zero_hbm, out_hbm,
                src_v, dst_v, rows0, rows1, acc, sem0, sem1):
    c = lax.axis_index("c")
    s = lax.axis_index("s")
    tid = c * NS + s
    nph = NB_E // 2
    pltpu.sync_copy(zero_hbm, acc.at[pl.ds(s * ROWS_T, ROWS_T)])
    plsc.subcore_barrier()

    for p in range(2):
        pltpu.sync_copy(srcb_hbm.at[tid, pl.ds(p * nph, nph)], src_v)
        pltpu.sync_copy(dstb_hbm.at[tid, pl.ds(p * nph, nph)], dst_v)
        pltpu.async_copy(h_hbm.at[src_v.at[0]], rows0, sem0)
        pltpu.async_copy(h_hbm.at[src_v.at[1]], rows1, sem1)

        def body(jj, carry):
            j0 = jj * 2
            pltpu.make_async_copy(h_hbm.at[src_v.at[j0]], rows0, sem0).wait()
            pltpu.sync_copy(rows0, acc.at[dst_v.at[j0]], add=True)
            pltpu.async_copy(h_hbm.at[src_v.at[j0 + 2]], rows0, sem0)
            pltpu.make_async_copy(h_hbm.at[src_v.at[j0]], rows1, sem1).wait()
            pltpu.sync_copy(rows1, acc.at[dst_v.at[j0 + 1]], add=True)
            pltpu.async_copy(h_hbm.at[src_v.at[j0 + 3]], rows1, sem1)
            return carry

        lax.fori_loop(0, nph // 2 - 1, body, 0)
        pltpu.make_async_copy(h_hbm.at[src_v.at[0]], rows0, sem0).wait()
        pltpu.sync_copy(rows0, acc.at[dst_v.at[nph - 2]], add=True)
        pltpu.make_async_copy(h_hbm.at[src_v.at[0]], rows1, sem1).wait()
        pltpu.sync_copy(rows1, acc.at[dst_v.at[nph - 1]], add=True)
    plsc.subcore_barrier()
    pltpu.sync_copy(acc.at[pl.ds(s * ROWS_T, ROWS_T)],
                    out_hbm.at[c, pl.ds(s * ROWS_T, ROWS_T)])


@functools.partial(
    pl.kernel,
    out_type=jax.ShapeDtypeStruct((2 * P, H), jnp.float32),
    mesh=_mesh,
    scratch_types=[
        pltpu.VMEM((NB_P, EB), jnp.int32),
        pltpu.VMEM((EB, H), jnp.float32),
        pltpu.VMEM((EB, H), jnp.float32),
        pltpu.SemaphoreType.DMA,
        pltpu.SemaphoreType.DMA,
    ],
)
def _pair_gather_kernel(z_hbm, pairs_hbm, out_hbm, idx_v, rows0, rows1,
                        sem0, sem1):
    c = lax.axis_index("c")
    s = lax.axis_index("s")
    tid = c * NS + s
    pltpu.sync_copy(pairs_hbm.at[tid], idx_v)
    base = tid * PB

    pltpu.async_copy(z_hbm.at[idx_v.at[0]], rows0, sem0)
    pltpu.async_copy(z_hbm.at[idx_v.at[1]], rows1, sem1)

    def body(jj, carry):
        j0 = jj * 2
        pltpu.make_async_copy(z_hbm.at[idx_v.at[j0]], rows0, sem0).wait()
        pltpu.sync_copy(rows0, out_hbm.at[pl.ds(base + j0 * EB, EB)])
        pltpu.async_copy(z_hbm.at[idx_v.at[j0 + 2]], rows0, sem0)
        pltpu.make_async_copy(z_hbm.at[idx_v.at[j0]], rows1, sem1).wait()
        pltpu.sync_copy(rows1, out_hbm.at[pl.ds(base + (j0 + 1) * EB, EB)])
        pltpu.async_copy(z_hbm.at[idx_v.at[j0 + 3]], rows1, sem1)
        return carry

    lax.fori_loop(0, NB_P // 2 - 1, body, 0)
    pltpu.make_async_copy(z_hbm.at[idx_v.at[0]], rows0, sem0).wait()
    pltpu.sync_copy(rows0, out_hbm.at[pl.ds(base + (NB_P - 2) * EB, EB)])
    pltpu.make_async_copy(z_hbm.at[idx_v.at[0]], rows1, sem1).wait()
    pltpu.sync_copy(rows1, out_hbm.at[pl.ds(base + (NB_P - 1) * EB, EB)])


def _mm_body(x_ref, w_ref, o_ref):
    o_ref[...] = jnp.dot(x_ref[...], w_ref[...],
                         preferred_element_type=jnp.float32)


_mm_tc = pl.pallas_call(
    _mm_body, out_shape=jax.ShapeDtypeStruct((N, H), jnp.float32))


def _scale_body(h_ref, degp_ref, hp_ref, dinv_ref):
    dinv = lax.rsqrt(degp_ref[0, :N, :] + degp_ref[1, :N, :] + 1.0)
    hp_ref[...] = h_ref[...] * dinv
    dinv_ref[...] = dinv


_scale_tc = pl.pallas_call(
    _scale_body,
    out_shape=(jax.ShapeDtypeStruct((N, H), jnp.float32),
               jax.ShapeDtypeStruct((N, 1), jnp.float32)))


def _fin_mm_body(a_ref, hp_ref, dinv_ref, b_ref, w_ref, o_ref):
    z = dinv_ref[...] * (a_ref[0, :N, :] + a_ref[1, :N, :] + hp_ref[...])
    z = jnp.maximum(z + b_ref[...], 0.0)
    o_ref[...] = jnp.dot(z, w_ref[...],
                         preferred_element_type=jnp.float32) * dinv_ref[...]


_fin_mm_tc = pl.pallas_call(
    _fin_mm_body, out_shape=jax.ShapeDtypeStruct((N, H), jnp.float32))


def _uv_body(a_ref, hp_ref, dinv_ref, b_ref, w_ref, o_ref):
    z2 = (dinv_ref[...] * (a_ref[0, :N, :] + a_ref[1, :N, :] + hp_ref[...])
          + b_ref[...])
    o_ref[0, ...] = jnp.dot(z2, w_ref[0], preferred_element_type=jnp.float32)


_uv_tc = pl.pallas_call(
    _uv_body,
    grid=(2,),
    in_specs=[
        pl.BlockSpec((NC, NPAD, H), lambda i: (0, 0, 0)),
        pl.BlockSpec((N, H), lambda i: (0, 0)),
        pl.BlockSpec((N, 1), lambda i: (0, 0)),
        pl.BlockSpec((1, H), lambda i: (0, 0)),
        pl.BlockSpec((1, H, H), lambda i: (i, 0, 0)),
    ],
    out_specs=pl.BlockSpec((1, N, H), lambda i: (i, 0, 0)),
    out_shape=jax.ShapeDtypeStruct((2, N, H), jnp.float32),
)


LB = 8192


def _link_body(us_ref, vd_ref, b1_ref, w2_ref, b2_ref, o_ref):
    hr = jnp.maximum(us_ref[...] + vd_ref[...] + b1_ref[...], 0.0)
    logit = jnp.sum(hr * w2_ref[...], axis=1, keepdims=True) + b2_ref[...]
    o_ref[...] = 1.0 / (1.0 + jnp.exp(-logit))


_link_tc = pl.pallas_call(
    _link_body,
    grid=(P // LB,),
    in_specs=[
        pl.BlockSpec((LB, H), lambda i: (i, 0)),
        pl.BlockSpec((LB, H), lambda i: (i + P // LB, 0)),
        pl.BlockSpec((1, H), lambda i: (0, 0)),
        pl.BlockSpec((1, H), lambda i: (0, 0)),
        pl.BlockSpec((1, 1), lambda i: (0, 0)),
    ],
    out_specs=pl.BlockSpec((LB, 1), lambda i: (i, 0)),
    out_shape=jax.ShapeDtypeStruct((P, 1), jnp.float32),
)


def kernel(x, edge_index_train, edge_pairs, W1, b1, W2, b2,
           lpW1, lpb1, lpW2, lpb2):
    src = edge_index_train[0].astype(jnp.int32)
    dst = edge_index_train[1].astype(jnp.int32)
    pad = EPAD - E
    pad_src = jnp.arange(pad, dtype=jnp.int32) % N
    pad_dst = N + (jnp.arange(pad, dtype=jnp.int32) % (NPAD - N))
    srcb = jnp.concatenate([src, pad_src]).reshape(NT, NB_E, EB)
    dstb = jnp.concatenate([dst, pad_dst]).reshape(NT, NB_E, EB)
    ep = edge_pairs.astype(jnp.int32)
    pairsb = jnp.concatenate([ep[0], ep[1] + N]).reshape(NT, NB_P, EB)

    ones_eb = jnp.ones((EB,), jnp.float32)
    zero_1d = jnp.zeros((ROWS_T,), jnp.float32)
    zero_2d = jnp.zeros((ROWS_T, H), jnp.float32)

    degp = _deg_kernel(dstb, ones_eb, zero_1d)
    degp = degp.reshape(NC, NPAD, 1)
    h1 = _mm_tc(x, W1)
    h1p, dinv = _scale_tc(h1, degp)

    agg1 = _agg_kernel(h1p, srcb, dstb, zero_2d)
    h2p = _fin_mm_tc(agg1, h1p, dinv, b1.reshape(1, H), W2)
    agg2 = _agg_kernel(h2p, srcb, dstb, zero_2d)

    wab = jnp.stack([lpW1[:H], lpW1[H:]])
    uv = _uv_tc(agg2, h2p, dinv, b2.reshape(1, H), wab)

    rows = _pair_gather_kernel(uv.reshape(2 * N, H), pairsb)
    out = _link_tc(rows, rows, lpb1.reshape(1, H), lpW2.reshape(1, H),
                   lpb2.reshape(1, 1))
    return out.reshape(P)


# speedup vs baseline: 25.5376x; 1.0158x over previous
import functools

import jax
import jax.numpy as jnp
from jax import lax
from jax.experimental import pallas as pl
from jax.experimental.pallas import tpu as pltpu
from jax.experimental.pallas import tpu_sc as plsc

N = 10000
D = 128
H = 128
E = 320000
P = 65536

NC = 2
NS = 16
NT = NC * NS
NPAD = 10240
ROWS_T = NPAD // NS
EB = 128
EPT = 10240
NB_E = EPT // EB
EPAD = NT * EPT
PB = (2 * P) // NT
NB_P = PB // EB

_mesh = plsc.VectorSubcoreMesh(core_axis_name="c", subcore_axis_name="s")


@functools.partial(
    pl.kernel,
    out_type=jax.ShapeDtypeStruct((NC, NPAD), jnp.float32),
    mesh=_mesh,
    scratch_types=[
        pltpu.VMEM((NB_E, EB), jnp.int32),
        pltpu.VMEM((EB,), jnp.float32),
        pltpu.VMEM_SHARED((NPAD,), jnp.float32),
    ],
)
def _deg_kernel(dstb_hbm, ones_hbm, zero_hbm, out_hbm, idx_v, ones_v, acc):
    c = lax.axis_index("c")
    s = lax.axis_index("s")
    tid = c * NS + s
    pltpu.sync_copy(zero_hbm, acc.at[pl.ds(s * ROWS_T, ROWS_T)])
    pltpu.sync_copy(ones_hbm, ones_v)
    pltpu.sync_copy(dstb_hbm.at[tid], idx_v)
    plsc.subcore_barrier()

    def body(j, carry):
        pltpu.sync_copy(ones_v, acc.at[idx_v.at[j]], add=True)
        return carry

    lax.fori_loop(0, NB_E, body, 0)
    plsc.subcore_barrier()
    pltpu.sync_copy(acc.at[pl.ds(s * ROWS_T, ROWS_T)],
                    out_hbm.at[c, pl.ds(s * ROWS_T, ROWS_T)])


@functools.partial(
    pl.kernel,
    out_type=jax.ShapeDtypeStruct((NC, NPAD, H), jnp.float32),
    mesh=_mesh,
    scratch_types=[
        pltpu.VMEM((NB_E // 2, EB), jnp.int32),
        pltpu.VMEM((NB_E // 2, EB), jnp.int32),
        pltpu.VMEM((EB, H), jnp.float32),
        pltpu.VMEM((EB, H), jnp.float32),
        pltpu.VMEM_SHARED((NPAD, H), jnp.float32),
        pltpu.SemaphoreType.DMA,
        pltpu.SemaphoreType.DMA,
    ],
)
def _agg_kernel(h_hbm, srcb_hbm, dstb_hbm, zero_hbm, out_hbm,
                src_v, dst_v, rows0, rows1, acc, sem0, sem1):
    c = lax.axis_index("c")
    s = lax.axis_index("s")
    tid = c * NS + s
    nph = NB_E // 2
    pltpu.sync_copy(zero_hbm, acc.at[pl.ds(s * ROWS_T, ROWS_T)])
    plsc.subcore_barrier()

    for p in range(2):
        pltpu.sync_copy(srcb_hbm.at[tid, pl.ds(p * nph, nph)], src_v)
        pltpu.sync_copy(dstb_hbm.at[tid, pl.ds(p * nph, nph)], dst_v)
        pltpu.async_copy(h_hbm.at[src_v.at[0]], rows0, sem0)
        pltpu.async_copy(h_hbm.at[src_v.at[1]], rows1, sem1)

        def body(jj, carry):
            j0 = jj * 2
            pltpu.make_async_copy(h_hbm.at[src_v.at[j0]], rows0, sem0).wait()
            pltpu.sync_copy(rows0, acc.at[dst_v.at[j0]], add=True)
            pltpu.async_copy(h_hbm.at[src_v.at[j0 + 2]], rows0, sem0)
            pltpu.make_async_copy(h_hbm.at[src_v.at[j0]], rows1, sem1).wait()
            pltpu.sync_copy(rows1, acc.at[dst_v.at[j0 + 1]], add=True)
            pltpu.async_copy(h_hbm.at[src_v.at[j0 + 3]], rows1, sem1)
            return carry

        lax.fori_loop(0, nph // 2 - 1, body, 0)
        pltpu.make_async_copy(h_hbm.at[src_v.at[0]], rows0, sem0).wait()
        pltpu.sync_copy(rows0, acc.at[dst_v.at[nph - 2]], add=True)
        pltpu.make_async_copy(h_hbm.at[src_v.at[0]], rows1, sem1).wait()
        pltpu.sync_copy(rows1, acc.at[dst_v.at[nph - 1]], add=True)
    plsc.subcore_barrier()
    pltpu.sync_copy(acc.at[pl.ds(s * ROWS_T, ROWS_T)],
                    out_hbm.at[c, pl.ds(s * ROWS_T, ROWS_T)])


@functools.partial(
    pl.kernel,
    out_type=jax.ShapeDtypeStruct((2 * P, H), jnp.float32),
    mesh=_mesh,
    scratch_types=[
        pltpu.VMEM((NB_P, EB), jnp.int32),
        pltpu.VMEM((EB, H), jnp.float32),
        pltpu.VMEM((EB, H), jnp.float32),
        pltpu.SemaphoreType.DMA,
        pltpu.SemaphoreType.DMA,
    ],
)
def _pair_gather_kernel(z_hbm, pairs_hbm, out_hbm, idx_v, rows0, rows1,
                        sem0, sem1):
    c = lax.axis_index("c")
    s = lax.axis_index("s")
    tid = c * NS + s
    pltpu.sync_copy(pairs_hbm.at[tid], idx_v)
    base = tid * PB

    pltpu.async_copy(z_hbm.at[idx_v.at[0]], rows0, sem0)
    pltpu.async_copy(z_hbm.at[idx_v.at[1]], rows1, sem1)

    def body(jj, carry):
        j0 = jj * 2
        pltpu.make_async_copy(z_hbm.at[idx_v.at[j0]], rows0, sem0).wait()
        pltpu.sync_copy(rows0, out_hbm.at[pl.ds(base + j0 * EB, EB)])
        pltpu.async_copy(z_hbm.at[idx_v.at[j0 + 2]], rows0, sem0)
        pltpu.make_async_copy(z_hbm.at[idx_v.at[j0]], rows1, sem1).wait()
        pltpu.sync_copy(rows1, out_hbm.at[pl.ds(base + (j0 + 1) * EB, EB)])
        pltpu.async_copy(z_hbm.at[idx_v.at[j0 + 3]], rows1, sem1)
        return carry

    lax.fori_loop(0, NB_P // 2 - 1, body, 0)
    pltpu.make_async_copy(z_hbm.at[idx_v.at[0]], rows0, sem0).wait()
    pltpu.sync_copy(rows0, out_hbm.at[pl.ds(base + (NB_P - 2) * EB, EB)])
    pltpu.make_async_copy(z_hbm.at[idx_v.at[0]], rows1, sem1).wait()
    pltpu.sync_copy(rows1, out_hbm.at[pl.ds(base + (NB_P - 1) * EB, EB)])


def _mm_body(x_ref, w_ref, o_ref):
    o_ref[...] = jnp.dot(x_ref[...], w_ref[...],
                         preferred_element_type=jnp.float32)


_mm_tc = pl.pallas_call(
    _mm_body, out_shape=jax.ShapeDtypeStruct((N, H), jnp.float32))


def _scale_body(h_ref, degp_ref, hp_ref, dinv_ref):
    dinv = lax.rsqrt(degp_ref[0, :N, :] + degp_ref[1, :N, :] + 1.0)
    hp_ref[...] = h_ref[...] * dinv
    dinv_ref[...] = dinv


_scale_tc = pl.pallas_call(
    _scale_body,
    out_shape=(jax.ShapeDtypeStruct((N, H), jnp.float32),
               jax.ShapeDtypeStruct((N, 1), jnp.float32)))


def _fin_mm_body(a_ref, hp_ref, dinv_ref, b_ref, w_ref, o_ref):
    z = dinv_ref[...] * (a_ref[0, :N, :] + a_ref[1, :N, :] + hp_ref[...])
    z = jnp.maximum(z + b_ref[...], 0.0)
    o_ref[...] = jnp.dot(z, w_ref[...],
                         preferred_element_type=jnp.float32) * dinv_ref[...]


_fin_mm_tc = pl.pallas_call(
    _fin_mm_body, out_shape=jax.ShapeDtypeStruct((N, H), jnp.float32))


def _uv_body(a_ref, hp_ref, dinv_ref, b_ref, w_ref, o_ref):
    z2 = (dinv_ref[...] * (a_ref[0, :N, :] + a_ref[1, :N, :] + hp_ref[...])
          + b_ref[...])
    o_ref[0, ...] = jnp.dot(z2, w_ref[0], preferred_element_type=jnp.float32)


_uv_tc = pl.pallas_call(
    _uv_body,
    grid=(2,),
    in_specs=[
        pl.BlockSpec((NC, NPAD, H), lambda i: (0, 0, 0)),
        pl.BlockSpec((N, H), lambda i: (0, 0)),
        pl.BlockSpec((N, 1), lambda i: (0, 0)),
        pl.BlockSpec((1, H), lambda i: (0, 0)),
        pl.BlockSpec((1, H, H), lambda i: (i, 0, 0)),
    ],
    out_specs=pl.BlockSpec((1, N, H), lambda i: (i, 0, 0)),
    out_shape=jax.ShapeDtypeStruct((2, N, H), jnp.float32),
)


LB = 8192


def _link_body(us_ref, vd_ref, b1_ref, w2_ref, b2_ref, o_ref):
    hr = jnp.maximum(us_ref[...] + vd_ref[...] + b1_ref[...], 0.0)
    logit = jnp.sum(hr * w2_ref[...], axis=1) + b2_ref[0, 0]
    o_ref[...] = 1.0 / (1.0 + jnp.exp(-logit))


_link_tc = pl.pallas_call(
    _link_body,
    grid=(P // LB,),
    in_specs=[
        pl.BlockSpec((LB, H), lambda i: (i, 0)),
        pl.BlockSpec((LB, H), lambda i: (i + P // LB, 0)),
        pl.BlockSpec((1, H), lambda i: (0, 0)),
        pl.BlockSpec((1, H), lambda i: (0, 0)),
        pl.BlockSpec((1, 1), lambda i: (0, 0)),
    ],
    out_specs=pl.BlockSpec((LB,), lambda i: (i,)),
    out_shape=jax.ShapeDtypeStruct((P,), jnp.float32),
)


def kernel(x, edge_index_train, edge_pairs, W1, b1, W2, b2,
           lpW1, lpb1, lpW2, lpb2):
    src = edge_index_train[0].astype(jnp.int32)
    dst = edge_index_train[1].astype(jnp.int32)
    pad = EPAD - E
    pad_src = jnp.arange(pad, dtype=jnp.int32) % N
    pad_dst = N + (jnp.arange(pad, dtype=jnp.int32) % (NPAD - N))
    srcb = jnp.concatenate([src, pad_src]).reshape(NT, NB_E, EB)
    dstb = jnp.concatenate([dst, pad_dst]).reshape(NT, NB_E, EB)
    ep = edge_pairs.astype(jnp.int32)
    pairsb = jnp.concatenate([ep[0], ep[1] + N]).reshape(NT, NB_P, EB)

    ones_eb = jnp.ones((EB,), jnp.float32)
    zero_1d = jnp.zeros((ROWS_T,), jnp.float32)
    zero_2d = jnp.zeros((ROWS_T, H), jnp.float32)

    h1 = _mm_tc(x, W1)
    degp = _deg_kernel(dstb, ones_eb, zero_1d)
    degp = degp.reshape(NC, NPAD, 1)
    h1p, dinv = _scale_tc(h1, degp)

    agg1 = _agg_kernel(h1p, srcb, dstb, zero_2d)
    h2p = _fin_mm_tc(agg1, h1p, dinv, b1.reshape(1, H), W2)
    agg2 = _agg_kernel(h2p, srcb, dstb, zero_2d)

    wab = jnp.stack([lpW1[:H], lpW1[H:]])
    uv = _uv_tc(agg2, h2p, dinv, b2.reshape(1, H), wab)

    rows = _pair_gather_kernel(uv.reshape(2 * N, H), pairsb)
    return _link_tc(rows, rows, lpb1.reshape(1, H), lpW2.reshape(1, H),
                    lpb2.reshape(1, 1))


# speedup vs baseline: 25.8584x; 1.0126x over previous
import functools

import jax
import jax.numpy as jnp
from jax import lax
from jax.experimental import pallas as pl
from jax.experimental.pallas import tpu as pltpu
from jax.experimental.pallas import tpu_sc as plsc

N = 10000
D = 128
H = 128
E = 320000
P = 65536

NC = 2
NS = 16
NT = NC * NS
NPAD = 10240
ROWS_T = NPAD // NS
EB = 128
EPT = 10240
NB_E = EPT // EB
EPAD = NT * EPT
PB = (2 * P) // NT
NB_P = PB // EB

_mesh = plsc.VectorSubcoreMesh(core_axis_name="c", subcore_axis_name="s")


@functools.partial(
    pl.kernel,
    out_type=jax.ShapeDtypeStruct((NC, NPAD), jnp.float32),
    mesh=_mesh,
    scratch_types=[
        pltpu.VMEM((NB_E, EB), jnp.int32),
        pltpu.VMEM((EB,), jnp.float32),
        pltpu.VMEM_SHARED((NPAD,), jnp.float32),
    ],
)
def _deg_kernel(dstb_hbm, ones_hbm, zero_hbm, out_hbm, idx_v, ones_v, acc):
    c = lax.axis_index("c")
    s = lax.axis_index("s")
    tid = c * NS + s
    pltpu.sync_copy(zero_hbm, acc.at[pl.ds(s * ROWS_T, ROWS_T)])
    pltpu.sync_copy(ones_hbm, ones_v)
    pltpu.sync_copy(dstb_hbm.at[tid], idx_v)
    plsc.subcore_barrier()

    def body(j, carry):
        pltpu.sync_copy(ones_v, acc.at[idx_v.at[j]], add=True)
        return carry

    lax.fori_loop(0, NB_E, body, 0)
    plsc.subcore_barrier()
    pltpu.sync_copy(acc.at[pl.ds(s * ROWS_T, ROWS_T)],
                    out_hbm.at[c, pl.ds(s * ROWS_T, ROWS_T)])


@functools.partial(
    pl.kernel,
    out_type=jax.ShapeDtypeStruct((NC, NPAD, H), jnp.float32),
    mesh=_mesh,
    scratch_types=[
        pltpu.VMEM((NB_E // 2, EB), jnp.int32),
        pltpu.VMEM((NB_E // 2, EB), jnp.int32),
        pltpu.VMEM((EB, H), jnp.float32),
        pltpu.VMEM((EB, H), jnp.float32),
        pltpu.VMEM_SHARED((NPAD, H), jnp.float32),
        pltpu.SemaphoreType.DMA,
        pltpu.SemaphoreType.DMA,
        pltpu.SemaphoreType.DMA,
    ],
)
def _agg_kernel(h_hbm, srcb_hbm, dstb_hbm, zero_hbm, out_hbm,
                src_v, dst_v, rows0, rows1, acc, sem0, sem1, semz):
    c = lax.axis_index("c")
    s = lax.axis_index("s")
    tid = c * NS + s
    nph = NB_E // 2
    zcp = pltpu.async_copy(zero_hbm, acc.at[pl.ds(s * ROWS_T, ROWS_T)], semz)
    first = True

    for p in range(2):
        pltpu.sync_copy(srcb_hbm.at[tid, pl.ds(p * nph, nph)], src_v)
        pltpu.sync_copy(dstb_hbm.at[tid, pl.ds(p * nph, nph)], dst_v)
        pltpu.async_copy(h_hbm.at[src_v.at[0]], rows0, sem0)
        pltpu.async_copy(h_hbm.at[src_v.at[1]], rows1, sem1)
        if first:
            zcp.wait()
            plsc.subcore_barrier()
            first = False

        def body(jj, carry):
            j0 = jj * 2
            pltpu.make_async_copy(h_hbm.at[src_v.at[j0]], rows0, sem0).wait()
            pltpu.sync_copy(rows0, acc.at[dst_v.at[j0]], add=True)
            pltpu.async_copy(h_hbm.at[src_v.at[j0 + 2]], rows0, sem0)
            pltpu.make_async_copy(h_hbm.at[src_v.at[j0]], rows1, sem1).wait()
            pltpu.sync_copy(rows1, acc.at[dst_v.at[j0 + 1]], add=True)
            pltpu.async_copy(h_hbm.at[src_v.at[j0 + 3]], rows1, sem1)
            return carry

        lax.fori_loop(0, nph // 2 - 1, body, 0)
        pltpu.make_async_copy(h_hbm.at[src_v.at[0]], rows0, sem0).wait()
        pltpu.sync_copy(rows0, acc.at[dst_v.at[nph - 2]], add=True)
        pltpu.make_async_copy(h_hbm.at[src_v.at[0]], rows1, sem1).wait()
        pltpu.sync_copy(rows1, acc.at[dst_v.at[nph - 1]], add=True)
    plsc.subcore_barrier()
    pltpu.sync_copy(acc.at[pl.ds(s * ROWS_T, ROWS_T)],
                    out_hbm.at[c, pl.ds(s * ROWS_T, ROWS_T)])


@functools.partial(
    pl.kernel,
    out_type=jax.ShapeDtypeStruct((2 * P, H), jnp.float32),
    mesh=_mesh,
    scratch_types=[
        pltpu.VMEM((NB_P, EB), jnp.int32),
        pltpu.VMEM((EB, H), jnp.float32),
        pltpu.VMEM((EB, H), jnp.float32),
        pltpu.SemaphoreType.DMA,
        pltpu.SemaphoreType.DMA,
    ],
)
def _pair_gather_kernel(z_hbm, pairs_hbm, out_hbm, idx_v, rows0, rows1,
                        sem0, sem1):
    c = lax.axis_index("c")
    s = lax.axis_index("s")
    tid = c * NS + s
    pltpu.sync_copy(pairs_hbm.at[tid], idx_v)
    base = tid * PB

    pltpu.async_copy(z_hbm.at[idx_v.at[0]], rows0, sem0)
    pltpu.async_copy(z_hbm.at[idx_v.at[1]], rows1, sem1)

    def body(jj, carry):
        j0 = jj * 2
        pltpu.make_async_copy(z_hbm.at[idx_v.at[j0]], rows0, sem0).wait()
        pltpu.sync_copy(rows0, out_hbm.at[pl.ds(base + j0 * EB, EB)])
        pltpu.async_copy(z_hbm.at[idx_v.at[j0 + 2]], rows0, sem0)
        pltpu.make_async_copy(z_hbm.at[idx_v.at[j0]], rows1, sem1).wait()
        pltpu.sync_copy(rows1, out_hbm.at[pl.ds(base + (j0 + 1) * EB, EB)])
        pltpu.async_copy(z_hbm.at[idx_v.at[j0 + 3]], rows1, sem1)
        return carry

    lax.fori_loop(0, NB_P // 2 - 1, body, 0)
    pltpu.make_async_copy(z_hbm.at[idx_v.at[0]], rows0, sem0).wait()
    pltpu.sync_copy(rows0, out_hbm.at[pl.ds(base + (NB_P - 2) * EB, EB)])
    pltpu.make_async_copy(z_hbm.at[idx_v.at[0]], rows1, sem1).wait()
    pltpu.sync_copy(rows1, out_hbm.at[pl.ds(base + (NB_P - 1) * EB, EB)])


def _mm_body(x_ref, w_ref, o_ref):
    o_ref[...] = jnp.dot(x_ref[...], w_ref[...],
                         preferred_element_type=jnp.float32)


_mm_tc = pl.pallas_call(
    _mm_body, out_shape=jax.ShapeDtypeStruct((N, H), jnp.float32))


def _scale_body(h_ref, degp_ref, hp_ref, dinv_ref):
    dinv = lax.rsqrt(degp_ref[0, :N, :] + degp_ref[1, :N, :] + 1.0)
    hp_ref[...] = h_ref[...] * dinv
    dinv_ref[...] = dinv


_scale_tc = pl.pallas_call(
    _scale_body,
    out_shape=(jax.ShapeDtypeStruct((N, H), jnp.float32),
               jax.ShapeDtypeStruct((N, 1), jnp.float32)))


def _fin_mm_body(a_ref, hp_ref, dinv_ref, b_ref, w_ref, o_ref):
    z = dinv_ref[...] * (a_ref[0, :N, :] + a_ref[1, :N, :] + hp_ref[...])
    z = jnp.maximum(z + b_ref[...], 0.0)
    o_ref[...] = jnp.dot(z, w_ref[...],
                         preferred_element_type=jnp.float32) * dinv_ref[...]


_fin_mm_tc = pl.pallas_call(
    _fin_mm_body, out_shape=jax.ShapeDtypeStruct((N, H), jnp.float32))


def _uv_body(a_ref, hp_ref, dinv_ref, b_ref, w_ref, o_ref):
    z2 = (dinv_ref[...] * (a_ref[0, :N, :] + a_ref[1, :N, :] + hp_ref[...])
          + b_ref[...])
    o_ref[0, ...] = jnp.dot(z2, w_ref[0], preferred_element_type=jnp.float32)


_uv_tc = pl.pallas_call(
    _uv_body,
    grid=(2,),
    in_specs=[
        pl.BlockSpec((NC, NPAD, H), lambda i: (0, 0, 0)),
        pl.BlockSpec((N, H), lambda i: (0, 0)),
        pl.BlockSpec((N, 1), lambda i: (0, 0)),
        pl.BlockSpec((1, H), lambda i: (0, 0)),
        pl.BlockSpec((1, H, H), lambda i: (i, 0, 0)),
    ],
    out_specs=pl.BlockSpec((1, N, H), lambda i: (i, 0, 0)),
    out_shape=jax.ShapeDtypeStruct((2, N, H), jnp.float32),
)


LB = 8192


def _link_body(us_ref, vd_ref, b1_ref, w2_ref, b2_ref, o_ref):
    hr = jnp.maximum(us_ref[...] + vd_ref[...] + b1_ref[...], 0.0)
    logit = jnp.sum(hr * w2_ref[...], axis=1) + b2_ref[0, 0]
    o_ref[...] = 1.0 / (1.0 + jnp.exp(-logit))


_link_tc = pl.pallas_call(
    _link_body,
    grid=(P // LB,),
    in_specs=[
        pl.BlockSpec((LB, H), lambda i: (i, 0)),
        pl.BlockSpec((LB, H), lambda i: (i + P // LB, 0)),
        pl.BlockSpec((1, H), lambda i: (0, 0)),
        pl.BlockSpec((1, H), lambda i: (0, 0)),
        pl.BlockSpec((1, 1), lambda i: (0, 0)),
    ],
    out_specs=pl.BlockSpec((LB,), lambda i: (i,)),
    out_shape=jax.ShapeDtypeStruct((P,), jnp.float32),
)


def kernel(x, edge_index_train, edge_pairs, W1, b1, W2, b2,
           lpW1, lpb1, lpW2, lpb2):
    src = edge_index_train[0].astype(jnp.int32)
    dst = edge_index_train[1].astype(jnp.int32)
    pad = EPAD - E
    pad_src = jnp.arange(pad, dtype=jnp.int32) % N
    pad_dst = N + (jnp.arange(pad, dtype=jnp.int32) % (NPAD - N))
    srcb = jnp.concatenate([src, pad_src]).reshape(NT, NB_E, EB)
    dstb = jnp.concatenate([dst, pad_dst]).reshape(NT, NB_E, EB)
    ep = edge_pairs.astype(jnp.int32)
    pairsb = jnp.concatenate([ep[0], ep[1] + N]).reshape(NT, NB_P, EB)

    ones_eb = jnp.ones((EB,), jnp.float32)
    zero_1d = jnp.zeros((ROWS_T,), jnp.float32)
    zero_2d = jnp.zeros((ROWS_T, H), jnp.float32)

    h1 = _mm_tc(x, W1)
    degp = _deg_kernel(dstb, ones_eb, zero_1d)
    degp = degp.reshape(NC, NPAD, 1)
    h1p, dinv = _scale_tc(h1, degp)

    agg1 = _agg_kernel(h1p, srcb, dstb, zero_2d)
    h2p = _fin_mm_tc(agg1, h1p, dinv, b1.reshape(1, H), W2)
    agg2 = _agg_kernel(h2p, srcb, dstb, zero_2d)

    wab = jnp.stack([lpW1[:H], lpW1[H:]])
    uv = _uv_tc(agg2, h2p, dinv, b2.reshape(1, H), wab)

    rows = _pair_gather_kernel(uv.reshape(2 * N, H), pairsb)
    return _link_tc(rows, rows, lpb1.reshape(1, H), lpW2.reshape(1, H),
                    lpb2.reshape(1, 1))
